# Initial kernel scaffold; baseline (speedup 1.0000x reference)
import jax
import jax.numpy as jnp
from jax.experimental import pallas as pl


def kernel(edge_index, edge_vals, in_feature, W, b):
    raise NotImplementedError("write your pallas kernel here")


# speedup vs baseline: 3.1504x; 3.1504x over previous
import functools

import jax
import jax.numpy as jnp
from jax import lax
from jax.experimental import pallas as pl
from jax.experimental.pallas import tpu as pltpu
from jax.experimental.pallas import tpu_sc as plsc

N = 10000
E = 320000
D = 128
NC = 2
NS = 16
K = 128
CH = 80
E_PAD = NC * NS * CH * K
RPT = 640
RPT_LAST = N - (NS - 1) * RPT
ZR = 80
MB = 1000


def _mm_body(x_ref, w_ref, o_ref):
    o_ref[...] = jnp.dot(x_ref[...], w_ref[...],
                         preferred_element_type=jnp.float32)


def _support(x, w):
    return pl.pallas_call(
        _mm_body,
        grid=(N // MB,),
        in_specs=[
            pl.BlockSpec((MB, D), lambda i: (i, 0)),
            pl.BlockSpec((D, D), lambda i: (0, 0)),
        ],
        out_specs=pl.BlockSpec((MB, D), lambda i: (i, 0)),
        out_shape=jax.ShapeDtypeStruct((N, D), jnp.float32),
    )(x, w)


def _merge_body(p_ref, b_ref, o_ref):
    o_ref[...] = p_ref[0] + p_ref[1] + b_ref[0]


def _merge(partials, b):
    return pl.pallas_call(
        _merge_body,
        grid=(N // MB,),
        in_specs=[
            pl.BlockSpec((NC, MB, D), lambda i: (0, i, 0)),
            pl.BlockSpec((1, D), lambda i: (0, 0)),
        ],
        out_specs=pl.BlockSpec((MB, D), lambda i: (i, 0)),
        out_shape=jax.ShapeDtypeStruct((N, D), jnp.float32),
    )(partials, b.reshape(1, D))


_mesh = plsc.VectorSubcoreMesh(
    core_axis_name="c", subcore_axis_name="s", num_cores=NC, num_subcores=NS)


@functools.partial(
    pl.kernel,
    out_type=jax.ShapeDtypeStruct((NC, N, D), jnp.float32),
    mesh=_mesh,
    scratch_types=[
        pltpu.VMEM((CH, K), jnp.int32),
        pltpu.VMEM((CH, K), jnp.int32),
        pltpu.VMEM((CH, K), jnp.float32),
        pltpu.VMEM((K, D), jnp.float32),
        pltpu.VMEM_SHARED((N, D), jnp.float32),
        pltpu.SemaphoreType.DMA,
        pltpu.SemaphoreType.DMA,
    ],
)
def _sc_spmm(src_hbm, dst_hbm, val_hbm, sup_hbm, out_hbm,
             src_v, dst_v, val_v, rows_v, acc_sh, sem_g, sem_s):
    c = lax.axis_index("c")
    s = lax.axis_index("s")

    zvec = jnp.zeros((16,), jnp.float32)

    def zfill(i, carry):
        for j in range(D // 16):
            rows_v[i, pl.ds(16 * j, 16)] = zvec
        return carry

    lax.fori_loop(0, ZR, zfill, 0)
    zsrc = rows_v.at[pl.ds(0, ZR)]

    @pl.when(s < NS - 1)
    def _():
        for r in range(RPT // ZR):
            pltpu.sync_copy(zsrc, acc_sh.at[pl.ds(s * RPT + r * ZR, ZR)])

    @pl.when(s == NS - 1)
    def _():
        for r in range(RPT_LAST // ZR):
            pltpu.sync_copy(
                zsrc, acc_sh.at[pl.ds((NS - 1) * RPT + r * ZR, ZR)])

    plsc.subcore_barrier()

    pltpu.sync_copy(src_hbm.at[c, s], src_v)
    pltpu.sync_copy(dst_hbm.at[c, s], dst_v)
    pltpu.sync_copy(val_hbm.at[c, s], val_v)

    def chunk(i, carry):
        pltpu.async_copy(sup_hbm.at[src_v.at[i]], rows_v, sem_g).wait()

        def scale(g, inner):
            vv = val_v[i, pl.ds(g * 16, 16)]
            for el in range(16):
                vb = jnp.full((16,), vv[el], jnp.float32)
                e = g * 16 + el
                for j in range(D // 16):
                    sl = pl.ds(16 * j, 16)
                    rows_v[e, sl] = rows_v[e, sl] * vb
            return inner

        lax.fori_loop(0, K // 16, scale, 0)
        pltpu.async_copy(rows_v, acc_sh.at[dst_v.at[i]], sem_s,
                         add=True).wait()
        return carry

    lax.fori_loop(0, CH, chunk, 0)
    plsc.subcore_barrier()

    @pl.when(s < NS - 1)
    def _():
        pltpu.sync_copy(acc_sh.at[pl.ds(s * RPT, RPT)],
                        out_hbm.at[c, pl.ds(s * RPT, RPT)])

    @pl.when(s == NS - 1)
    def _():
        pltpu.sync_copy(acc_sh.at[pl.ds((NS - 1) * RPT, RPT_LAST)],
                        out_hbm.at[c, pl.ds((NS - 1) * RPT, RPT_LAST)])


def kernel(edge_index, edge_vals, in_feature, W, b):
    edge_index = edge_index.astype(jnp.int32)
    pad = E_PAD - E
    src = jnp.pad(edge_index[1], (0, pad)).reshape(NC, NS, CH, K)
    dst = jnp.pad(edge_index[0], (0, pad)).reshape(NC, NS, CH, K)
    val = jnp.pad(edge_vals, (0, pad)).reshape(NC, NS, CH, K)
    sup = _support(in_feature, W)
    partials = _sc_spmm(src, dst, val, sup)
    return _merge(partials, b)


# speedup vs baseline: 3.7940x; 1.2043x over previous
import functools

import jax
import jax.numpy as jnp
from jax import lax
from jax.experimental import pallas as pl
from jax.experimental.pallas import tpu as pltpu
from jax.experimental.pallas import tpu_sc as plsc

N = 10000
E = 320000
D = 128
NC = 2
NS = 16
K = 128
CH = 80
E_PAD = NC * NS * CH * K
SG = 8
QG = CH // SG
RPT = 640
RPT_LAST = N - (NS - 1) * RPT
ZR = 80
MB = 1000


def _mm_body(x_ref, w_ref, o_ref):
    o_ref[...] = jnp.dot(x_ref[...], w_ref[...],
                         preferred_element_type=jnp.float32)


def _support(x, w):
    return pl.pallas_call(
        _mm_body,
        grid=(N // MB,),
        in_specs=[
            pl.BlockSpec((MB, D), lambda i: (i, 0)),
            pl.BlockSpec((D, D), lambda i: (0, 0)),
        ],
        out_specs=pl.BlockSpec((MB, D), lambda i: (i, 0)),
        out_shape=jax.ShapeDtypeStruct((N, D), jnp.float32),
    )(x, w)


def _merge_body(p_ref, b_ref, o_ref):
    o_ref[...] = p_ref[0] + p_ref[1] + b_ref[0]


def _merge(partials, b):
    return pl.pallas_call(
        _merge_body,
        grid=(N // MB,),
        in_specs=[
            pl.BlockSpec((NC, MB, D), lambda i: (0, i, 0)),
            pl.BlockSpec((1, D), lambda i: (0, 0)),
        ],
        out_specs=pl.BlockSpec((MB, D), lambda i: (i, 0)),
        out_shape=jax.ShapeDtypeStruct((N, D), jnp.float32),
    )(partials, b.reshape(1, D))


_mesh = plsc.VectorSubcoreMesh(
    core_axis_name="c", subcore_axis_name="s", num_cores=NC, num_subcores=NS)


@functools.partial(
    pl.kernel,
    out_type=jax.ShapeDtypeStruct((NC, N, D), jnp.float32),
    mesh=_mesh,
    scratch_types=[
        pltpu.VMEM((CH, K), jnp.int32),
        pltpu.VMEM((2, SG, K), jnp.int32),
        pltpu.VMEM((2, SG, K), jnp.float32),
        pltpu.VMEM((2, K, D), jnp.float32),
        pltpu.VMEM_SHARED((N, D), jnp.float32),
        [pltpu.SemaphoreType.DMA] * 2,
        [pltpu.SemaphoreType.DMA] * 2,
        [pltpu.SemaphoreType.DMA] * 2,
    ],
)
def _sc_spmm(src_hbm, dst_hbm, val_hbm, sup_hbm, out_hbm,
             src_v, dst_sb, val_sb, rows_v, acc_sh, sem_g, sem_s, sem_i):
    c = lax.axis_index("c")
    s = lax.axis_index("s")

    zvec = jnp.zeros((16,), jnp.float32)

    def zfill(i, carry):
        for j in range(D // 16):
            rows_v[0, i, pl.ds(16 * j, 16)] = zvec
        return carry

    lax.fori_loop(0, ZR, zfill, 0)
    zsrc = rows_v.at[0, pl.ds(0, ZR)]

    @pl.when(s < NS - 1)
    def _():
        for r in range(RPT // ZR):
            pltpu.sync_copy(zsrc, acc_sh.at[pl.ds(s * RPT + r * ZR, ZR)])

    @pl.when(s == NS - 1)
    def _():
        for r in range(RPT_LAST // ZR):
            pltpu.sync_copy(
                zsrc, acc_sh.at[pl.ds((NS - 1) * RPT + r * ZR, ZR)])

    plsc.subcore_barrier()

    pltpu.sync_copy(src_hbm.at[c, s], src_v)

    def start_idx(q, p):
        sl = pl.ds(q * SG, SG)
        pltpu.async_copy(dst_hbm.at[c, s, sl], dst_sb.at[p], sem_i[p])
        pltpu.async_copy(val_hbm.at[c, s, sl], val_sb.at[p], sem_i[p])

    def wait_idx(p):
        pltpu.make_async_copy(
            dst_hbm.at[c, s, pl.ds(0, SG)], dst_sb.at[p], sem_i[p]).wait()
        pltpu.make_async_copy(
            val_hbm.at[c, s, pl.ds(0, SG)], val_sb.at[p], sem_i[p]).wait()

    def start_gather(i, b):
        pltpu.async_copy(sup_hbm.at[src_v.at[i]], rows_v.at[b], sem_g[b])

    def wait_gather(b):
        pltpu.make_async_copy(
            sup_hbm.at[src_v.at[0]], rows_v.at[b], sem_g[b]).wait()

    def start_scatter(b, p, j):
        pltpu.async_copy(rows_v.at[b], acc_sh.at[dst_sb.at[p, j]],
                         sem_s[b], add=True)

    def wait_scatter(b):
        pltpu.make_async_copy(
            rows_v.at[b], acc_sh.at[dst_sb.at[0, 0]], sem_s[b]).wait()

    def scale_buf(b, p, j):
        def scale(g, inner):
            vv = val_sb[p, j, pl.ds(g * 16, 16)]
            for el in range(16):
                vb = jnp.full((16,), vv[el], jnp.float32)
                e = g * 16 + el
                for jj in range(D // 16):
                    sl = pl.ds(16 * jj, 16)
                    rows_v[b, e, sl] = rows_v[b, e, sl] * vb
            return inner

        lax.fori_loop(0, K // 16, scale, 0)

    start_idx(0, 0)
    start_gather(0, 0)

    def group(q2, carry):
        for qq in range(2):
            q = q2 * 2 + qq
            for jj in range(SG // 2):
                for b in range(2):
                    j = jj * 2 + b
                    i = q * SG + j
                    b2 = 1 - b
                    wait_gather(b)
                    if jj == 0 and b == 0:
                        @pl.when(q > 0)
                        def _():
                            wait_scatter(b2)

                        @pl.when(q < QG - 1)
                        def _():
                            start_idx(q + 1, 1 - qq)

                        wait_idx(qq)
                    else:
                        wait_scatter(b2)
                    if qq == 1 and jj == SG // 2 - 1 and b == 1:
                        @pl.when(q2 < QG // 2 - 1)
                        def _():
                            start_gather(i + 1, b2)
                    else:
                        start_gather(i + 1, b2)
                    scale_buf(b, qq, j)
                    start_scatter(b, qq, j)
        return carry

    lax.fori_loop(0, QG // 2, group, 0)
    wait_scatter((CH - 1) % 2)
    plsc.subcore_barrier()

    @pl.when(s < NS - 1)
    def _():
        pltpu.sync_copy(acc_sh.at[pl.ds(s * RPT, RPT)],
                        out_hbm.at[c, pl.ds(s * RPT, RPT)])

    @pl.when(s == NS - 1)
    def _():
        pltpu.sync_copy(acc_sh.at[pl.ds((NS - 1) * RPT, RPT_LAST)],
                        out_hbm.at[c, pl.ds((NS - 1) * RPT, RPT_LAST)])


def kernel(edge_index, edge_vals, in_feature, W, b):
    edge_index = edge_index.astype(jnp.int32)
    pad = E_PAD - E
    src = jnp.pad(edge_index[1], (0, pad)).reshape(NC, NS, CH, K)
    dst = jnp.pad(edge_index[0], (0, pad)).reshape(NC, NS, CH, K)
    val = jnp.pad(edge_vals, (0, pad)).reshape(NC, NS, CH, K)
    sup = _support(in_feature, W)
    partials = _sc_spmm(src, dst, val, sup)
    return _merge(partials, b)


# speedup vs baseline: 10.8512x; 2.8601x over previous
import functools

import jax
import jax.numpy as jnp
from jax import lax
from jax.experimental import pallas as pl
from jax.experimental.pallas import tpu as pltpu
from jax.experimental.pallas import tpu_sc as plsc

N = 10000
E = 320000
D = 128
NC = 2
NS = 16
K = 128
CH = 80
E_PAD = NC * NS * CH * K
SG = 8
QG = CH // SG
RPT = 640
RPT_LAST = N - (NS - 1) * RPT
ZR = 80
MB = 1000


def _mm_body(x_ref, w_ref, o_ref):
    o_ref[...] = jnp.dot(x_ref[...], w_ref[...],
                         preferred_element_type=jnp.float32)


def _support(x, w):
    return pl.pallas_call(
        _mm_body,
        grid=(N // MB,),
        in_specs=[
            pl.BlockSpec((MB, D), lambda i: (i, 0)),
            pl.BlockSpec((D, D), lambda i: (0, 0)),
        ],
        out_specs=pl.BlockSpec((MB, D), lambda i: (i, 0)),
        out_shape=jax.ShapeDtypeStruct((N, D), jnp.float32),
    )(x, w)


def _merge_body(p_ref, b_ref, o_ref):
    o_ref[...] = p_ref[0] + p_ref[1] + b_ref[0]


def _merge(partials, b):
    return pl.pallas_call(
        _merge_body,
        grid=(N // MB,),
        in_specs=[
            pl.BlockSpec((NC, MB, D), lambda i: (0, i, 0)),
            pl.BlockSpec((1, D), lambda i: (0, 0)),
        ],
        out_specs=pl.BlockSpec((MB, D), lambda i: (i, 0)),
        out_shape=jax.ShapeDtypeStruct((N, D), jnp.float32),
    )(partials, b.reshape(1, D))


_mesh = plsc.VectorSubcoreMesh(
    core_axis_name="c", subcore_axis_name="s", num_cores=NC, num_subcores=NS)


@functools.partial(
    pl.kernel,
    out_type=jax.ShapeDtypeStruct((NC, N, D), jnp.float32),
    mesh=_mesh,
    scratch_types=[
        pltpu.VMEM((CH, K), jnp.int32),
        pltpu.VMEM((2, SG, K), jnp.int32),
        pltpu.VMEM((2, SG, K), jnp.float32),
        pltpu.VMEM((2, K, D), jnp.float32),
        pltpu.VMEM_SHARED((N, D), jnp.float32),
        [pltpu.SemaphoreType.DMA] * 2,
        [pltpu.SemaphoreType.DMA] * 2,
        [pltpu.SemaphoreType.DMA] * 2,
    ],
)
def _sc_spmm(src_hbm, dst_hbm, val_hbm, sup_hbm, out_hbm,
             src_v, dst_sb, val_sb, rows_v, acc_sh, sem_g, sem_s, sem_i):
    c = lax.axis_index("c")
    s = lax.axis_index("s")

    zvec = jnp.zeros((16,), jnp.float32)

    def zfill(i, carry):
        for j in range(D // 16):
            rows_v[0, i, pl.ds(16 * j, 16)] = zvec
        return carry

    lax.fori_loop(0, ZR, zfill, 0)
    zsrc = rows_v.at[0, pl.ds(0, ZR)]

    @pl.when(s < NS - 1)
    def _():
        for r in range(RPT // ZR):
            pltpu.sync_copy(zsrc, acc_sh.at[pl.ds(s * RPT + r * ZR, ZR)])

    @pl.when(s == NS - 1)
    def _():
        for r in range(RPT_LAST // ZR):
            pltpu.sync_copy(
                zsrc, acc_sh.at[pl.ds((NS - 1) * RPT + r * ZR, ZR)])

    plsc.subcore_barrier()

    pltpu.sync_copy(src_hbm.at[c, s], src_v)

    def start_idx(q, p):
        sl = pl.ds(q * SG, SG)
        pltpu.async_copy(dst_hbm.at[c, s, sl], dst_sb.at[p], sem_i[p])
        pltpu.async_copy(val_hbm.at[c, s, sl], val_sb.at[p], sem_i[p])

    def wait_idx(p):
        pltpu.make_async_copy(
            dst_hbm.at[c, s, pl.ds(0, SG)], dst_sb.at[p], sem_i[p]).wait()
        pltpu.make_async_copy(
            val_hbm.at[c, s, pl.ds(0, SG)], val_sb.at[p], sem_i[p]).wait()

    def start_gather(i, b):
        pltpu.async_copy(sup_hbm.at[src_v.at[i]], rows_v.at[b], sem_g[b])

    def wait_gather(b):
        pltpu.make_async_copy(
            sup_hbm.at[src_v.at[0]], rows_v.at[b], sem_g[b]).wait()

    def start_scatter(b, p, j):
        pltpu.async_copy(rows_v.at[b], acc_sh.at[dst_sb.at[p, j]],
                         sem_s[b], add=True)

    def wait_scatter(b):
        pltpu.make_async_copy(
            rows_v.at[b], acc_sh.at[dst_sb.at[0, 0]], sem_s[b]).wait()

    def scale_buf(b, p, j):
        def scale(g, inner):
            vv = val_sb[p, j, pl.ds(g * 16, 16)]
            for el in range(16):
                vb = jnp.full((16,), vv[el], jnp.float32)
                e = g * 16 + el
                for jj in range(D // 16):
                    sl = pl.ds(16 * jj, 16)
                    rows_v[b, e, sl] = rows_v[b, e, sl] * vb
            return inner

        lax.fori_loop(0, K // 16, scale, 0)

    start_idx(0, 0)
    start_gather(0, 0)

    def group(q2, carry):
        for qq in range(2):
            q = q2 * 2 + qq
            for jj in range(SG // 2):
                for b in range(2):
                    j = jj * 2 + b
                    i = q * SG + j
                    b2 = 1 - b
                    wait_gather(b)
                    if jj == 0 and b == 0:
                        @pl.when(q > 0)
                        def _():
                            wait_scatter(b2)

                        @pl.when(q < QG - 1)
                        def _():
                            start_idx(q + 1, 1 - qq)

                        wait_idx(qq)
                    else:
                        wait_scatter(b2)
                    if qq == 1 and jj == SG // 2 - 1 and b == 1:
                        @pl.when(q2 < QG // 2 - 1)
                        def _():
                            start_gather(i + 1, b2)
                    else:
                        start_gather(i + 1, b2)
                    scale_buf(b, qq, j)
                    start_scatter(b, qq, j)
        return carry

    lax.fori_loop(0, QG // 2, group, 0)
    wait_scatter((CH - 1) % 2)
    plsc.subcore_barrier()

    @pl.when(s < NS - 1)
    def _():
        pltpu.sync_copy(acc_sh.at[pl.ds(s * RPT, RPT)],
                        out_hbm.at[c, pl.ds(s * RPT, RPT)])

    @pl.when(s == NS - 1)
    def _():
        pltpu.sync_copy(acc_sh.at[pl.ds((NS - 1) * RPT, RPT_LAST)],
                        out_hbm.at[c, pl.ds((NS - 1) * RPT, RPT_LAST)])


def kernel(edge_index, edge_vals, in_feature, W, b):
    edge_index = edge_index.astype(jnp.int32)
    pad = E_PAD - E
    idx_pad = jnp.arange(pad, dtype=jnp.int32) % N
    src = jnp.concatenate([edge_index[1], idx_pad]).reshape(NC, NS, CH, K)
    dst = jnp.concatenate([edge_index[0], idx_pad]).reshape(NC, NS, CH, K)
    val = jnp.pad(edge_vals, (0, pad)).reshape(NC, NS, CH, K)
    sup = _support(in_feature, W)
    partials = _sc_spmm(src, dst, val, sup)
    return _merge(partials, b)


# speedup vs baseline: 10.8533x; 1.0002x over previous
import functools

import jax
import jax.numpy as jnp
from jax import lax
from jax.experimental import pallas as pl
from jax.experimental.pallas import tpu as pltpu
from jax.experimental.pallas import tpu_sc as plsc

N = 10000
E = 320000
D = 128
NC = 2
NS = 16
K = 112
CH = 90
E_PAD = NC * NS * CH * K
SG = 5
QG = CH // SG
MQ = QG // 3
RPT = 640
RPT_LAST = N - (NS - 1) * RPT
ZR = 80
MB = 1000


def _mm_body(x_ref, w_ref, o_ref):
    o_ref[...] = jnp.dot(x_ref[...], w_ref[...],
                         preferred_element_type=jnp.float32)


def _support(x, w):
    return pl.pallas_call(
        _mm_body,
        grid=(N // MB,),
        in_specs=[
            pl.BlockSpec((MB, D), lambda i: (i, 0)),
            pl.BlockSpec((D, D), lambda i: (0, 0)),
        ],
        out_specs=pl.BlockSpec((MB, D), lambda i: (i, 0)),
        out_shape=jax.ShapeDtypeStruct((N, D), jnp.float32),
    )(x, w)


def _merge_body(p_ref, b_ref, o_ref):
    o_ref[...] = p_ref[0] + p_ref[1] + b_ref[0]


def _merge(partials, b):
    return pl.pallas_call(
        _merge_body,
        grid=(N // MB,),
        in_specs=[
            pl.BlockSpec((NC, MB, D), lambda i: (0, i, 0)),
            pl.BlockSpec((1, D), lambda i: (0, 0)),
        ],
        out_specs=pl.BlockSpec((MB, D), lambda i: (i, 0)),
        out_shape=jax.ShapeDtypeStruct((N, D), jnp.float32),
    )(partials, b.reshape(1, D))


_mesh = plsc.VectorSubcoreMesh(
    core_axis_name="c", subcore_axis_name="s", num_cores=NC, num_subcores=NS)


@functools.partial(
    pl.kernel,
    out_type=jax.ShapeDtypeStruct((NC, N, D), jnp.float32),
    mesh=_mesh,
    compiler_params=pltpu.CompilerParams(use_tc_tiling_on_sc=False),
    scratch_types=[
        pltpu.VMEM((3, SG, K), jnp.int32),
        pltpu.VMEM((3, SG, K), jnp.int32),
        pltpu.VMEM((3, SG, K), jnp.float32),
        pltpu.VMEM((3, K, D), jnp.float32),
        pltpu.VMEM_SHARED((N, D), jnp.float32),
        [pltpu.SemaphoreType.DMA] * 3,
        [pltpu.SemaphoreType.DMA] * 3,
        [pltpu.SemaphoreType.DMA] * 3,
    ],
)
def _sc_spmm(src_hbm, dst_hbm, val_hbm, sup_hbm, out_hbm,
             src_sb, dst_sb, val_sb, rows_v, acc_sh, sem_g, sem_s, sem_i):
    c = lax.axis_index("c")
    s = lax.axis_index("s")

    zvec = jnp.zeros((16,), jnp.float32)

    def zfill(i, carry):
        for j in range(D // 16):
            rows_v[0, i, pl.ds(16 * j, 16)] = zvec
        return carry

    lax.fori_loop(0, ZR, zfill, 0)
    zsrc = rows_v.at[0, pl.ds(0, ZR)]

    @pl.when(s < NS - 1)
    def _():
        for r in range(RPT // ZR):
            pltpu.sync_copy(zsrc, acc_sh.at[pl.ds(s * RPT + r * ZR, ZR)])

    @pl.when(s == NS - 1)
    def _():
        for r in range(RPT_LAST // ZR):
            pltpu.sync_copy(
                zsrc, acc_sh.at[pl.ds((NS - 1) * RPT + r * ZR, ZR)])

    plsc.subcore_barrier()

    def start_idx(q, p):
        sl = pl.ds(q * SG, SG)
        pltpu.async_copy(src_hbm.at[c, s, sl], src_sb.at[p], sem_i[p])
        pltpu.async_copy(dst_hbm.at[c, s, sl], dst_sb.at[p], sem_i[p])
        pltpu.async_copy(val_hbm.at[c, s, sl], val_sb.at[p], sem_i[p])

    def wait_idx(p):
        sl = pl.ds(0, SG)
        pltpu.make_async_copy(
            src_hbm.at[c, s, sl], src_sb.at[p], sem_i[p]).wait()
        pltpu.make_async_copy(
            dst_hbm.at[c, s, sl], dst_sb.at[p], sem_i[p]).wait()
        pltpu.make_async_copy(
            val_hbm.at[c, s, sl], val_sb.at[p], sem_i[p]).wait()

    def start_gather(b, p, j):
        pltpu.async_copy(sup_hbm.at[src_sb.at[p, j]], rows_v.at[b],
                         sem_g[b])

    def wait_gather(b):
        pltpu.make_async_copy(
            sup_hbm.at[src_sb.at[0, 0]], rows_v.at[b], sem_g[b]).wait()

    def start_scatter(b, p, j):
        pltpu.async_copy(rows_v.at[b], acc_sh.at[dst_sb.at[p, j]],
                         sem_s[b], add=True)

    def wait_scatter(b):
        pltpu.make_async_copy(
            rows_v.at[b], acc_sh.at[dst_sb.at[0, 0]], sem_s[b]).wait()

    def scale_buf(b, p, j):
        def scale(g, inner):
            vv = val_sb[p, j, pl.ds(g * 16, 16)]
            for el in range(16):
                vb = jnp.full((16,), vv[el], jnp.float32)
                e = g * 16 + el
                for jj in range(D // 16):
                    sl = pl.ds(16 * jj, 16)
                    rows_v[b, e, sl] = rows_v[b, e, sl] * vb
            return inner

        lax.fori_loop(0, K // 16, scale, 0)

    start_idx(0, 0)
    wait_idx(0)
    start_gather(0, 0, 0)

    def macro(m, carry):
        for qq in range(3):
            q = m * 3 + qq
            for j in range(SG):
                b = (qq * SG + j) % 3
                b2 = (qq * SG + j + 1) % 3
                if j == 0:
                    if qq == 2:
                        @pl.when(m < MQ - 1)
                        def _():
                            start_idx(q + 1, 0)
                    else:
                        start_idx(q + 1, qq + 1)
                wait_gather(b)
                if qq == 0 and j < 2:
                    @pl.when(m > 0)
                    def _():
                        wait_scatter(b2)
                else:
                    wait_scatter(b2)
                if j == SG - 1:
                    if qq == 2:
                        @pl.when(m < MQ - 1)
                        def _():
                            wait_idx(0)
                            start_gather(b2, 0, 0)
                    else:
                        wait_idx(qq + 1)
                        start_gather(b2, qq + 1, 0)
                else:
                    start_gather(b2, qq, j + 1)
                scale_buf(b, qq, j)
                start_scatter(b, qq, j)
        return carry

    lax.fori_loop(0, MQ, macro, 0)
    wait_scatter((CH - 2) % 3)
    wait_scatter((CH - 1) % 3)
    plsc.subcore_barrier()

    @pl.when(s < NS - 1)
    def _():
        pltpu.sync_copy(acc_sh.at[pl.ds(s * RPT, RPT)],
                        out_hbm.at[c, pl.ds(s * RPT, RPT)])

    @pl.when(s == NS - 1)
    def _():
        pltpu.sync_copy(acc_sh.at[pl.ds((NS - 1) * RPT, RPT_LAST)],
                        out_hbm.at[c, pl.ds((NS - 1) * RPT, RPT_LAST)])


def kernel(edge_index, edge_vals, in_feature, W, b):
    edge_index = edge_index.astype(jnp.int32)
    pad = E_PAD - E
    idx_pad = jnp.arange(pad, dtype=jnp.int32) % N
    src = jnp.concatenate([edge_index[1], idx_pad]).reshape(NC, NS, CH, K)
    dst = jnp.concatenate([edge_index[0], idx_pad]).reshape(NC, NS, CH, K)
    val = jnp.pad(edge_vals, (0, pad)).reshape(NC, NS, CH, K)
    sup = _support(in_feature, W)
    partials = _sc_spmm(src, dst, val, sup)
    return _merge(partials, b)


# speedup vs baseline: 10.9053x; 1.0048x over previous
import functools

import jax
import jax.numpy as jnp
from jax import lax
from jax.experimental import pallas as pl
from jax.experimental.pallas import tpu as pltpu
from jax.experimental.pallas import tpu_sc as plsc

N = 10000
E = 320000
D = 128
NC = 2
NS = 16
K = 112
CH = 90
E_PAD = NC * NS * CH * K
SG = 5
QG = CH // SG
MQ = QG // 3
RPT = 640
RPT_LAST = N - (NS - 1) * RPT
ZR = 80
MB = 1000


def _mm_body(x_ref, w_ref, o_ref):
    o_ref[...] = jnp.dot(x_ref[...], w_ref[...],
                         preferred_element_type=jnp.float32)


def _support(x, w):
    return pl.pallas_call(
        _mm_body,
        grid=(N // MB,),
        in_specs=[
            pl.BlockSpec((MB, D), lambda i: (i, 0)),
            pl.BlockSpec((D, D), lambda i: (0, 0)),
        ],
        out_specs=pl.BlockSpec((MB, D), lambda i: (i, 0)),
        out_shape=jax.ShapeDtypeStruct((N, D), jnp.float32),
    )(x, w)


def _merge_body(p_ref, b_ref, o_ref):
    o_ref[...] = p_ref[0] + p_ref[1] + b_ref[0]


def _merge(partials, b):
    return pl.pallas_call(
        _merge_body,
        grid=(N // MB,),
        in_specs=[
            pl.BlockSpec((NC, MB, D), lambda i: (0, i, 0)),
            pl.BlockSpec((1, D), lambda i: (0, 0)),
        ],
        out_specs=pl.BlockSpec((MB, D), lambda i: (i, 0)),
        out_shape=jax.ShapeDtypeStruct((N, D), jnp.float32),
    )(partials, b.reshape(1, D))


_mesh = plsc.VectorSubcoreMesh(
    core_axis_name="c", subcore_axis_name="s", num_cores=NC, num_subcores=NS)


@functools.partial(
    pl.kernel,
    out_type=jax.ShapeDtypeStruct((NC, N, D), jnp.float32),
    mesh=_mesh,
    compiler_params=pltpu.CompilerParams(use_tc_tiling_on_sc=False),
    scratch_types=[
        pltpu.VMEM((3, SG, K), jnp.int32),
        pltpu.VMEM((3, SG, K), jnp.int32),
        pltpu.VMEM((3, SG, K), jnp.float32),
        pltpu.VMEM((3, K, D), jnp.float32),
        pltpu.VMEM_SHARED((N, D), jnp.float32),
        [pltpu.SemaphoreType.DMA] * 3,
        [pltpu.SemaphoreType.DMA] * 3,
        [pltpu.SemaphoreType.DMA] * 3,
    ],
)
def _sc_spmm(src_hbm, dst_hbm, val_hbm, sup_hbm, out_hbm,
             src_sb, dst_sb, val_sb, rows_v, acc_sh, sem_g, sem_s, sem_i):
    c = lax.axis_index("c")
    s = lax.axis_index("s")

    def start_idx(q, p):
        sl = pl.ds(q * SG, SG)
        pltpu.async_copy(src_hbm.at[c, s, sl], src_sb.at[p], sem_i[p])
        pltpu.async_copy(dst_hbm.at[c, s, sl], dst_sb.at[p], sem_i[p])
        pltpu.async_copy(val_hbm.at[c, s, sl], val_sb.at[p], sem_i[p])

    def wait_idx(p):
        sl = pl.ds(0, SG)
        pltpu.make_async_copy(
            src_hbm.at[c, s, sl], src_sb.at[p], sem_i[p]).wait()
        pltpu.make_async_copy(
            dst_hbm.at[c, s, sl], dst_sb.at[p], sem_i[p]).wait()
        pltpu.make_async_copy(
            val_hbm.at[c, s, sl], val_sb.at[p], sem_i[p]).wait()

    def start_gather(b, p, j):
        h = K // 2
        pltpu.async_copy(sup_hbm.at[src_sb.at[p, j, pl.ds(0, h)]],
                         rows_v.at[b, pl.ds(0, h)], sem_g[b])
        pltpu.async_copy(sup_hbm.at[src_sb.at[p, j, pl.ds(h, h)]],
                         rows_v.at[b, pl.ds(h, h)], sem_g[b])

    def wait_gather(b):
        h = K // 2
        pltpu.make_async_copy(
            sup_hbm.at[src_sb.at[0, 0, pl.ds(0, h)]],
            rows_v.at[b, pl.ds(0, h)], sem_g[b]).wait()
        pltpu.make_async_copy(
            sup_hbm.at[src_sb.at[0, 0, pl.ds(0, h)]],
            rows_v.at[b, pl.ds(h, h)], sem_g[b]).wait()

    def start_scatter(b, p, j):
        pltpu.async_copy(rows_v.at[b], acc_sh.at[dst_sb.at[p, j]],
                         sem_s[b], add=True)

    def wait_scatter(b):
        pltpu.make_async_copy(
            rows_v.at[b], acc_sh.at[dst_sb.at[0, 0]], sem_s[b]).wait()

    def scale_buf(b, p, j):
        def scale(g, inner):
            vv = val_sb[p, j, pl.ds(g * 16, 16)]
            for el in range(16):
                vb = jnp.full((16,), vv[el], jnp.float32)
                e = g * 16 + el
                for jj in range(D // 16):
                    sl = pl.ds(16 * jj, 16)
                    rows_v[b, e, sl] = rows_v[b, e, sl] * vb
            return inner

        lax.fori_loop(0, K // 16, scale, 0)

    start_idx(0, 0)
    zvec = jnp.zeros((16,), jnp.float32)

    def zfill(i, carry):
        for j in range(D // 16):
            rows_v[1, i, pl.ds(16 * j, 16)] = zvec
        return carry

    lax.fori_loop(0, ZR, zfill, 0)
    wait_idx(0)
    start_gather(0, 0, 0)
    zsrc = rows_v.at[1, pl.ds(0, ZR)]

    @pl.when(s < NS - 1)
    def _():
        for r in range(RPT // ZR):
            pltpu.sync_copy(zsrc, acc_sh.at[pl.ds(s * RPT + r * ZR, ZR)])

    @pl.when(s == NS - 1)
    def _():
        for r in range(RPT_LAST // ZR):
            pltpu.sync_copy(
                zsrc, acc_sh.at[pl.ds((NS - 1) * RPT + r * ZR, ZR)])

    plsc.subcore_barrier()

    def macro(m, carry):
        for qq in range(3):
            q = m * 3 + qq
            for j in range(SG):
                b = (qq * SG + j) % 3
                b2 = (qq * SG + j + 1) % 3
                if j == 0:
                    if qq == 2:
                        @pl.when(m < MQ - 1)
                        def _():
                            start_idx(q + 1, 0)
                    else:
                        start_idx(q + 1, qq + 1)
                wait_gather(b)
                if qq == 0 and j < 2:
                    @pl.when(m > 0)
                    def _():
                        wait_scatter(b2)
                else:
                    wait_scatter(b2)
                if j == SG - 1:
                    if qq == 2:
                        @pl.when(m < MQ - 1)
                        def _():
                            wait_idx(0)
                            start_gather(b2, 0, 0)
                    else:
                        wait_idx(qq + 1)
                        start_gather(b2, qq + 1, 0)
                else:
                    start_gather(b2, qq, j + 1)
                scale_buf(b, qq, j)
                start_scatter(b, qq, j)
        return carry

    lax.fori_loop(0, MQ, macro, 0)
    wait_scatter((CH - 2) % 3)
    wait_scatter((CH - 1) % 3)
    plsc.subcore_barrier()

    @pl.when(s < NS - 1)
    def _():
        pltpu.sync_copy(acc_sh.at[pl.ds(s * RPT, RPT)],
                        out_hbm.at[c, pl.ds(s * RPT, RPT)])

    @pl.when(s == NS - 1)
    def _():
        pltpu.sync_copy(acc_sh.at[pl.ds((NS - 1) * RPT, RPT_LAST)],
                        out_hbm.at[c, pl.ds((NS - 1) * RPT, RPT_LAST)])


def kernel(edge_index, edge_vals, in_feature, W, b):
    edge_index = edge_index.astype(jnp.int32)
    pad = E_PAD - E
    idx_pad = jnp.arange(pad, dtype=jnp.int32) % N
    src = jnp.concatenate([edge_index[1], idx_pad]).reshape(NC, NS, CH, K)
    dst = jnp.concatenate([edge_index[0], idx_pad]).reshape(NC, NS, CH, K)
    val = jnp.pad(edge_vals, (0, pad)).reshape(NC, NS, CH, K)
    sup = _support(in_feature, W)
    partials = _sc_spmm(src, dst, val, sup)
    return _merge(partials, b)
